# baseline (speedup 1.0000x reference)
import functools

import jax
import jax.numpy as jnp
from jax import lax
from jax.experimental import pallas as pl
from jax.experimental.pallas import tpu as pltpu
from jax.experimental.pallas import tpu_sc as plsc

_SEQ = 8192
_DIM = 1024
_BSZ = 4
_NC = 2
_NS = 16
_NW = _NC * _NS
_ROWS_PER_W = _SEQ // _NW


@functools.partial(
    pl.kernel,
    out_type=jax.ShapeDtypeStruct((_BSZ, _SEQ, _DIM), jnp.float32),
    mesh=plsc.VectorSubcoreMesh(core_axis_name="c", subcore_axis_name="s"),
    scratch_types=[pltpu.SemaphoreType.DMA],
)
def _bcast_kernel(table_hbm, out_hbm, sem):
    wid = lax.axis_index("s") * _NC + lax.axis_index("c")
    base = wid * _ROWS_PER_W
    copies = [
        pltpu.async_copy(
            table_hbm.at[pl.ds(base, _ROWS_PER_W)],
            out_hbm.at[b, pl.ds(base, _ROWS_PER_W)],
            sem)
        for b in range(_BSZ)
    ]
    for d in copies:
        d.wait()


def kernel(inputs, table):
    del inputs
    return _bcast_kernel(table)
